# baseline (device time: 19290 ns/iter reference)
import jax
import jax.numpy as jnp
from jax import lax
from jax.experimental import pallas as pl
from jax.experimental.pallas import tpu as pltpu

NCF = 16


def kernel(x, dy):
    k, d = x.shape
    _, f = dy.shape
    half = d // 2
    q = half // 2
    fc = f // NCF

    def body(x_ref, dy_ref, out_ref, send_ref, recv_ref,
             sem_ys, sem_yr, sem_xs, sem_xr, xrdy):
        my_x = lax.axis_index("x")
        my_y = lax.axis_index("y")
        my_z = lax.axis_index("z")
        other = 1 - my_y
        y_nbr = (my_x, other, my_z)
        x_nbr = (1 - my_x, my_y, my_z)

        barrier_sem = pltpu.get_barrier_semaphore()
        pl.semaphore_signal(
            barrier_sem, inc=1, device_id=y_nbr,
            device_id_type=pl.DeviceIdType.MESH,
        )
        pl.semaphore_signal(
            xrdy, inc=1, device_id=x_nbr,
            device_id_type=pl.DeviceIdType.MESH,
        )

        x_sq = x_ref[:, pl.ds(other * half + my_x * q, q)].astype(jnp.bfloat16)
        x_keep = x_ref[:, pl.ds(my_y * half, half)].astype(jnp.bfloat16)

        for c in range(NCF):
            cs = pl.ds(c * fc, fc)
            dyb_c = dy_ref[:, cs].astype(jnp.bfloat16)
            ps = lax.dot_general(
                x_sq, dyb_c, (((0,), (0,)), ((), ())),
                preferred_element_type=jnp.float32,
            )
            send_ref[c, :, :] = ps.astype(jnp.bfloat16)
            pk = lax.dot_general(
                x_keep, dyb_c, (((0,), (0,)), ((), ())),
                preferred_element_type=jnp.float32,
            )
            out_ref[:, cs] = pk

        pl.semaphore_wait(barrier_sem, 1)

        rdma_y = []
        for c in range(NCF):
            r = pltpu.make_async_remote_copy(
                src_ref=send_ref.at[c],
                dst_ref=recv_ref.at[0, c],
                send_sem=sem_ys.at[c],
                recv_sem=sem_yr.at[c],
                device_id=y_nbr,
                device_id_type=pl.DeviceIdType.MESH,
            )
            r.start()
            rdma_y.append(r)

        pl.semaphore_wait(xrdy, 1)

        yq = pl.ds(my_x * q, q)
        rdma_x = []
        for c in range(NCF):
            rdma_y[c].wait_recv()
            r = pltpu.make_async_remote_copy(
                src_ref=recv_ref.at[0, c],
                dst_ref=recv_ref.at[1, c],
                send_sem=sem_xs.at[c],
                recv_sem=sem_xr.at[c],
                device_id=x_nbr,
                device_id_type=pl.DeviceIdType.MESH,
            )
            r.start()
            rdma_x.append(r)
            cs = pl.ds(c * fc, fc)
            out_ref[yq, cs] = (
                out_ref[yq, cs] + recv_ref[0, c, :, :].astype(jnp.float32)
            )

        xq = pl.ds((1 - my_x) * q, q)
        for c in range(NCF):
            rdma_x[c].wait_recv()
            cs = pl.ds(c * fc, fc)
            out_ref[xq, cs] = (
                out_ref[xq, cs] + recv_ref[1, c, :, :].astype(jnp.float32)
            )

        for c in range(NCF):
            rdma_y[c].wait_send()
            rdma_x[c].wait_send()

    return pl.pallas_call(
        body,
        out_shape=jax.ShapeDtypeStruct((half, f), jnp.float32),
        in_specs=[
            pl.BlockSpec(memory_space=pltpu.VMEM),
            pl.BlockSpec(memory_space=pltpu.VMEM),
        ],
        out_specs=pl.BlockSpec(memory_space=pltpu.VMEM),
        scratch_shapes=[
            pltpu.VMEM((NCF, q, fc), jnp.bfloat16),
            pltpu.VMEM((2, NCF, q, fc), jnp.bfloat16),
            pltpu.SemaphoreType.DMA((NCF,)),
            pltpu.SemaphoreType.DMA((NCF,)),
            pltpu.SemaphoreType.DMA((NCF,)),
            pltpu.SemaphoreType.DMA((NCF,)),
            pltpu.SemaphoreType.REGULAR,
        ],
        compiler_params=pltpu.CompilerParams(collective_id=0),
    )(x, dy)


# device time: 17581 ns/iter; 1.0972x vs baseline; 1.0972x over previous
import functools

import jax
import jax.numpy as jnp
from jax import lax
from jax.experimental import pallas as pl
from jax.experimental.pallas import tpu as pltpu

NCF = 8


def kernel(x, dy):
    k, d = x.shape
    _, f = dy.shape
    half = d // 2
    q = half // 2
    fc = f // NCF

    def body(x_ref, dy_ref, out_ref, send_ref, recv_ref,
             sem_ys, sem_yr, sem_xs, sem_xr, xrdy):
        my_x = lax.axis_index("x")
        my_y = lax.axis_index("y")
        my_z = lax.axis_index("z")
        other = 1 - my_y
        y_nbr = (my_x, other, my_z)
        x_nbr = (1 - my_x, my_y, my_z)

        barrier_sem = pltpu.get_barrier_semaphore()
        pl.semaphore_signal(
            barrier_sem, inc=1, device_id=y_nbr,
            device_id_type=pl.DeviceIdType.MESH,
        )
        pl.semaphore_signal(
            xrdy, inc=1, device_id=x_nbr,
            device_id_type=pl.DeviceIdType.MESH,
        )

        x_sq = x_ref[:, pl.ds(other * half + my_x * q, q)].astype(jnp.bfloat16)
        x_keep = x_ref[:, pl.ds(my_y * half, half)].astype(jnp.bfloat16)

        NPRE = 2
        for c in range(NPRE):
            cs = pl.ds(c * fc, fc)
            dyb_c = dy_ref[:, cs].astype(jnp.bfloat16)
            ps = lax.dot_general(
                x_sq, dyb_c, (((0,), (0,)), ((), ())),
                preferred_element_type=jnp.float32,
            )
            send_ref[c, :, :] = ps.astype(jnp.bfloat16)

        pl.semaphore_wait(barrier_sem, 1)

        rdma_y = []
        for c in range(NCF):
            cs = pl.ds(c * fc, fc)
            dyb_c = dy_ref[:, cs].astype(jnp.bfloat16)
            if c >= NPRE:
                ps = lax.dot_general(
                    x_sq, dyb_c, (((0,), (0,)), ((), ())),
                    preferred_element_type=jnp.float32,
                )
                send_ref[c, :, :] = ps.astype(jnp.bfloat16)
            r = pltpu.make_async_remote_copy(
                src_ref=send_ref.at[c],
                dst_ref=recv_ref.at[0, c],
                send_sem=sem_ys.at[c],
                recv_sem=sem_yr.at[c],
                device_id=y_nbr,
                device_id_type=pl.DeviceIdType.MESH,
            )
            r.start()
            rdma_y.append(r)
            pk = lax.dot_general(
                x_keep, dyb_c, (((0,), (0,)), ((), ())),
                preferred_element_type=jnp.float32,
            )
            out_ref[:, cs] = pk

        pl.semaphore_wait(xrdy, 1)

        yq = pl.ds(my_x * q, q)
        rdma_x = []
        for c in range(NCF):
            rdma_y[c].wait_recv()
            r = pltpu.make_async_remote_copy(
                src_ref=recv_ref.at[0, c],
                dst_ref=recv_ref.at[1, c],
                send_sem=sem_xs.at[c],
                recv_sem=sem_xr.at[c],
                device_id=x_nbr,
                device_id_type=pl.DeviceIdType.MESH,
            )
            r.start()
            rdma_x.append(r)
            cs = pl.ds(c * fc, fc)
            out_ref[yq, cs] = (
                out_ref[yq, cs] + recv_ref[0, c, :, :].astype(jnp.float32)
            )

        xq = pl.ds((1 - my_x) * q, q)
        for c in range(NCF):
            rdma_x[c].wait_recv()
            cs = pl.ds(c * fc, fc)
            out_ref[xq, cs] = (
                out_ref[xq, cs] + recv_ref[1, c, :, :].astype(jnp.float32)
            )

        for c in range(NCF):
            rdma_y[c].wait_send()
            rdma_x[c].wait_send()

    return pl.pallas_call(
        body,
        out_shape=jax.ShapeDtypeStruct((half, f), jnp.float32),
        in_specs=[
            pl.BlockSpec(memory_space=pltpu.VMEM),
            pl.BlockSpec(memory_space=pltpu.VMEM),
        ],
        out_specs=pl.BlockSpec(memory_space=pltpu.VMEM),
        scratch_shapes=[
            pltpu.VMEM((NCF, q, fc), jnp.bfloat16),
            pltpu.VMEM((2, NCF, q, fc), jnp.bfloat16),
            pltpu.SemaphoreType.DMA((NCF,)),
            pltpu.SemaphoreType.DMA((NCF,)),
            pltpu.SemaphoreType.DMA((NCF,)),
            pltpu.SemaphoreType.DMA((NCF,)),
            pltpu.SemaphoreType.REGULAR,
        ],
        compiler_params=pltpu.CompilerParams(collective_id=0),
    )(x, dy)
